# baseline (speedup 1.0000x reference)
import functools

import jax
import jax.numpy as jnp
from jax import lax
from jax.experimental import pallas as pl
from jax.experimental.pallas import tpu as pltpu
from jax.experimental.pallas import tpu_sc as plsc

NUM_EMBEDDINGS = 2048
EMBEDDING_DIM = 256
COMMITMENT_COST = 0.25

_N = 16
_T = 2048
_ROWS = _N * _T
_BLK = 512
_TB = _T // _BLK
_NBLK = _ROWS // _BLK

_NC = 2
_NS = 16
_NW = _NC * _NS
_RPW = _ROWS // _NW
_SUB = 128
_NCHUNK = _RPW // _SUB


def _dist_argmin_kernel(x_ref, w2_ref, wsq_ref, oh_ref, qo_ref, loss_ref,
                        perp_ref, cnt_ref, acc_ref):
    i = pl.program_id(0)
    xb = x_ref[0]
    w2 = w2_ref[...]

    @pl.when(i == 0)
    def _():
        cnt_ref[...] = jnp.zeros((8, NUM_EMBEDDINGS), jnp.float32)
        acc_ref[0] = 0.0

    mm2 = jax.lax.dot_general(
        xb, w2, (((0,), (1,)), ((), ())),
        preferred_element_type=jnp.float32)
    xsq = jnp.sum(xb * xb, axis=0)[:, None]
    d = (xsq + wsq_ref[...]) - mm2

    minval = jnp.min(d, axis=1, keepdims=True)
    oh = jnp.where(d == minval, 1.0, 0.0)
    oh_ref[...] = oh

    qo_ref[0] = jax.lax.dot_general(
        w2, oh, (((0,), (1,)), ((), ())),
        preferred_element_type=jnp.float32) * 0.5

    cnt_ref[...] += jax.lax.dot_general(
        jnp.ones((8, _BLK), jnp.float32), oh, (((1,), (0,)), ((), ())),
        preferred_element_type=jnp.float32)
    acc_ref[0] += jnp.sum(minval)

    @pl.when(i == _NBLK - 1)
    def _():
        loss_ref[...] = jnp.full(
            (1, 1), (1.0 + COMMITMENT_COST) / (_ROWS * EMBEDDING_DIM)) \
            * acc_ref[0]
        p = cnt_ref[0:1, :] * (1.0 / _ROWS)
        perp_ref[...] = jnp.exp(
            -jnp.sum(p * jnp.log(p + 1e-10), keepdims=True))


def _dist_argmin(inputs, W2, wsq):
    return pl.pallas_call(
        _dist_argmin_kernel,
        grid=(_NBLK,),
        in_specs=[
            pl.BlockSpec((1, EMBEDDING_DIM, _BLK),
                         lambda i: (i // _TB, 0, i % _TB)),
            pl.BlockSpec((NUM_EMBEDDINGS, EMBEDDING_DIM), lambda i: (0, 0)),
            pl.BlockSpec((1, NUM_EMBEDDINGS), lambda i: (0, 0)),
        ],
        out_specs=[
            pl.BlockSpec((_BLK, NUM_EMBEDDINGS), lambda i: (i, 0)),
            pl.BlockSpec((1, EMBEDDING_DIM, _BLK),
                         lambda i: (i // _TB, 0, i % _TB)),
            pl.BlockSpec((1, 1), lambda i: (0, 0)),
            pl.BlockSpec((1, 1), lambda i: (0, 0)),
        ],
        out_shape=[
            jax.ShapeDtypeStruct((_ROWS, NUM_EMBEDDINGS), jnp.float32),
            jax.ShapeDtypeStruct((_N, EMBEDDING_DIM, _T), jnp.float32),
            jax.ShapeDtypeStruct((1, 1), jnp.float32),
            jax.ShapeDtypeStruct((1, 1), jnp.float32),
        ],
        scratch_shapes=[
            pltpu.VMEM((8, NUM_EMBEDDINGS), jnp.float32),
            pltpu.SMEM((1,), jnp.float32),
        ],
    )(inputs, W2, wsq)


_NBUF = 3


def _sc_gather_body(idx_hbm, w_hbm, quant_hbm, *refs):
    idx_c = refs[:_NCHUNK]
    rows = refs[_NCHUNK:_NCHUNK + _NBUF]
    gsems = refs[_NCHUNK + _NBUF:_NCHUNK + 2 * _NBUF]
    wsems = refs[_NCHUNK + 2 * _NBUF:_NCHUNK + 3 * _NBUF]
    wid = lax.axis_index("s") * _NC + lax.axis_index("c")
    base = wid * _RPW
    for c in range(_NCHUNK):
        pltpu.sync_copy(idx_hbm.at[wid, c], idx_c[c])

    gcp = [None] * _NCHUNK
    wcp = [None] * _NCHUNK
    for c in range(min(_NBUF, _NCHUNK)):
        gcp[c] = pltpu.async_copy(w_hbm.at[idx_c[c]], rows[c % _NBUF],
                                  gsems[c % _NBUF])
    for c in range(_NCHUNK):
        b = c % _NBUF
        if c > 0 and c - 1 + _NBUF < _NCHUNK:
            wcp[c - 1].wait()
            gcp[c - 1 + _NBUF] = pltpu.async_copy(
                w_hbm.at[idx_c[c - 1 + _NBUF]], rows[(c - 1) % _NBUF],
                gsems[(c - 1) % _NBUF])
        gcp[c].wait()
        wcp[c] = pltpu.async_copy(
            rows[b], quant_hbm.at[pl.ds(base + c * _SUB, _SUB)], wsems[b])
    for c in range(_NCHUNK):
        if c + _NBUF >= _NCHUNK:
            wcp[c].wait()


def _sc_gather(encoding_indices, W):
    mesh = plsc.VectorSubcoreMesh(core_axis_name="c", subcore_axis_name="s")
    fn = functools.partial(
        pl.kernel,
        mesh=mesh,
        out_type=jax.ShapeDtypeStruct((_ROWS, EMBEDDING_DIM), jnp.float32),
        scratch_types=(
            [pltpu.VMEM((_SUB,), jnp.int32)] * _NCHUNK
            + [pltpu.VMEM((_SUB, EMBEDDING_DIM), jnp.float32)] * _NBUF
            + [pltpu.SemaphoreType.DMA] * (2 * _NBUF)
        ),
    )(_sc_gather_body)
    return fn(encoding_indices, W)


def _transpose_kernel(q_ref, out_ref):
    out_ref[0] = q_ref[...].T


def _transpose(quantized):
    return pl.pallas_call(
        _transpose_kernel,
        grid=(_N, _TB),
        in_specs=[
            pl.BlockSpec((_BLK, EMBEDDING_DIM),
                         lambda n, tb: (n * _TB + tb, 0)),
        ],
        out_specs=pl.BlockSpec((1, EMBEDDING_DIM, _BLK),
                               lambda n, tb: (n, 0, tb)),
        out_shape=jax.ShapeDtypeStruct((_N, EMBEDDING_DIM, _T), jnp.float32),
    )(quantized)


def kernel(inputs, W):
    W2 = W + W
    wsq = jnp.sum(W ** 2, axis=1)[None, :]
    encodings, quantized_out, loss, perp = _dist_argmin(inputs, W2, wsq)

    loss = loss.reshape(())
    perplexity = perp.reshape(())
    reset_ratio = jnp.zeros((1,), dtype=jnp.float32)

    return (loss, quantized_out, perplexity, reset_ratio, encodings)


# speedup vs baseline: 1.1110x; 1.1110x over previous
import jax
import jax.numpy as jnp
from jax.experimental import pallas as pl
from jax.experimental.pallas import tpu as pltpu

NUM_EMBEDDINGS = 2048
EMBEDDING_DIM = 256
COMMITMENT_COST = 0.25

_N = 16
_T = 2048
_ROWS = _N * _T
_BLK = 1024
_TB = _T // _BLK
_NBLK = _ROWS // _BLK


def _vq_kernel(x_ref, w2_ref, wsq_ref, oh_ref, qo_ref, loss_ref,
               perp_ref, cnt_ref, acc_ref):
    i = pl.program_id(0)
    xb = x_ref[0]
    w2 = w2_ref[...]

    @pl.when(i == 0)
    def _():
        cnt_ref[...] = jnp.zeros((8, NUM_EMBEDDINGS), jnp.float32)
        acc_ref[0] = 0.0

    mm2 = jax.lax.dot_general(
        xb, w2, (((0,), (1,)), ((), ())),
        preferred_element_type=jnp.float32)
    xsq = jnp.sum(xb * xb, axis=0)[:, None]
    d = (xsq + wsq_ref[...]) - mm2

    minval = jnp.min(d, axis=1, keepdims=True)
    oh = jnp.where(d == minval, 1.0, 0.0)
    oh_ref[...] = oh

    qo_ref[0] = jax.lax.dot_general(
        w2, oh, (((0,), (1,)), ((), ())),
        preferred_element_type=jnp.float32) * 0.5

    cnt_ref[...] += jax.lax.dot_general(
        jnp.ones((8, _BLK), jnp.float32), oh, (((1,), (0,)), ((), ())),
        preferred_element_type=jnp.float32)
    acc_ref[0] += jnp.sum(minval)

    @pl.when(i == _NBLK - 1)
    def _():
        loss_ref[...] = jnp.full(
            (1, 1), (1.0 + COMMITMENT_COST) / (_ROWS * EMBEDDING_DIM)) \
            * acc_ref[0]
        p = cnt_ref[0:1, :] * (1.0 / _ROWS)
        perp_ref[...] = jnp.exp(
            -jnp.sum(p * jnp.log(p + 1e-10), keepdims=True))


def _vq_fused(inputs, W2, wsq):
    return pl.pallas_call(
        _vq_kernel,
        grid=(_NBLK,),
        in_specs=[
            pl.BlockSpec((1, EMBEDDING_DIM, _BLK),
                         lambda i: (i // _TB, 0, i % _TB)),
            pl.BlockSpec((NUM_EMBEDDINGS, EMBEDDING_DIM), lambda i: (0, 0)),
            pl.BlockSpec((1, NUM_EMBEDDINGS), lambda i: (0, 0)),
        ],
        out_specs=[
            pl.BlockSpec((_BLK, NUM_EMBEDDINGS), lambda i: (i, 0)),
            pl.BlockSpec((1, EMBEDDING_DIM, _BLK),
                         lambda i: (i // _TB, 0, i % _TB)),
            pl.BlockSpec((1, 1), lambda i: (0, 0)),
            pl.BlockSpec((1, 1), lambda i: (0, 0)),
        ],
        out_shape=[
            jax.ShapeDtypeStruct((_ROWS, NUM_EMBEDDINGS), jnp.float32),
            jax.ShapeDtypeStruct((_N, EMBEDDING_DIM, _T), jnp.float32),
            jax.ShapeDtypeStruct((1, 1), jnp.float32),
            jax.ShapeDtypeStruct((1, 1), jnp.float32),
        ],
        scratch_shapes=[
            pltpu.VMEM((8, NUM_EMBEDDINGS), jnp.float32),
            pltpu.SMEM((1,), jnp.float32),
        ],
    )(inputs, W2, wsq)


def kernel(inputs, W):
    W2 = W + W
    wsq = jnp.sum(W ** 2, axis=1)[None, :]
    encodings, quantized_out, loss, perp = _vq_fused(inputs, W2, wsq)

    loss = loss.reshape(())
    perplexity = perp.reshape(())
    reset_ratio = jnp.zeros((1,), dtype=jnp.float32)

    return (loss, quantized_out, perplexity, reset_ratio, encodings)
